# Initial kernel scaffold; baseline (speedup 1.0000x reference)
import jax
import jax.numpy as jnp
from jax.experimental import pallas as pl


def kernel(x, indices, values):
    raise NotImplementedError("write your pallas kernel here")


# speedup vs baseline: 12.6483x; 12.6483x over previous
import functools

import jax
import jax.numpy as jnp
from jax import lax
from jax.experimental import pallas as pl
from jax.experimental.pallas import tpu as pltpu
from jax.experimental.pallas import tpu_sc as plsc

IN_SIZE = 65536
OUT_SIZE = 65536
CONN = 16
BATCH = 64

NC = 2
NS = 16
NW = NC * NS
ROWS_PER_W = OUT_SIZE // NW
CH = 64
NCHUNK = ROWS_PER_W // CH
G = CH * CONN // 128

_mesh = plsc.VectorSubcoreMesh(core_axis_name="c", subcore_axis_name="s")


@functools.partial(
    pl.kernel,
    mesh=_mesh,
    out_type=jax.ShapeDtypeStruct((OUT_SIZE, BATCH), jnp.float32),
    scratch_types=[
        pltpu.VMEM((G, 128), jnp.int32),
        pltpu.VMEM((CH, CONN), jnp.float32),
        pltpu.VMEM((CH * CONN, BATCH), jnp.float32),
        pltpu.VMEM((CH, BATCH), jnp.float32),
        pltpu.SemaphoreType.DMA,
    ],
    compiler_params=pltpu.CompilerParams(use_tc_tiling_on_sc=False),
)
def _spmm_sc(xt_hbm, idx_hbm, val_hbm, out_hbm, idx_v, val_v, rows_v, out_v,
             sem):
    wid = lax.axis_index("s") * NC + lax.axis_index("c")
    row_base = wid * ROWS_PER_W

    def chunk_body(i, carry):
        o0 = pl.multiple_of(row_base + i * CH, CH)
        g0 = pl.multiple_of(o0 * CONN // 128, CH * CONN // 128)
        pltpu.sync_copy(idx_hbm.at[pl.ds(g0, G)], idx_v)
        pltpu.sync_copy(val_hbm.at[pl.ds(o0, CH)], val_v)
        for j in range(G):
            pltpu.async_copy(xt_hbm.at[idx_v.at[j]],
                             rows_v.at[pl.ds(j * 128, 128)], sem).wait()

        def col_body(c, inner_carry):
            accs = [jnp.zeros((16,), jnp.float32) for _ in range(BATCH // 16)]
            vv = val_v[c, :]
            for k in range(CONN):
                s = vv[k]
                for j in range(BATCH // 16):
                    accs[j] = accs[j] + s * rows_v[c * CONN + k,
                                                   pl.ds(j * 16, 16)]
            for j in range(BATCH // 16):
                out_v[c, pl.ds(j * 16, 16)] = accs[j]
            return inner_carry

        lax.fori_loop(0, CH, col_body, 0)
        pltpu.sync_copy(out_v, out_hbm.at[pl.ds(o0, CH)])
        return carry

    lax.fori_loop(0, NCHUNK, chunk_body, 0)


def kernel(x, indices, values):
    xt = x.T
    idx2d = indices[1].reshape(CONN, OUT_SIZE).T.reshape(-1, 128)
    valt = values.reshape(CONN, OUT_SIZE).T
    outt = _spmm_sc(xt, idx2d, valt)
    return outt.T


# speedup vs baseline: 20.9398x; 1.6555x over previous
import functools

import jax
import jax.numpy as jnp
from jax import lax
from jax.experimental import pallas as pl
from jax.experimental.pallas import tpu as pltpu
from jax.experimental.pallas import tpu_sc as plsc

IN_SIZE = 65536
OUT_SIZE = 65536
CONN = 16
BATCH = 64

NC = 2
NS = 16
NW = NC * NS
ROWS_PER_W = OUT_SIZE // NW
UNIT = 16
NUNIT = ROWS_PER_W // UNIT
GPER = UNIT * CONN // 128
GROWS_PER_W = ROWS_PER_W * CONN // 128

_mesh = plsc.VectorSubcoreMesh(core_axis_name="c", subcore_axis_name="s")


@functools.partial(
    pl.kernel,
    mesh=_mesh,
    out_type=jax.ShapeDtypeStruct((OUT_SIZE, BATCH), jnp.float32),
    scratch_types=[
        pltpu.VMEM((GROWS_PER_W, 128), jnp.int32),
        pltpu.VMEM((ROWS_PER_W, CONN), jnp.float32),
        pltpu.VMEM((2, UNIT * CONN, BATCH), jnp.float32),
        pltpu.VMEM((2, UNIT, BATCH), jnp.float32),
        pltpu.SemaphoreType.DMA,
        pltpu.SemaphoreType.DMA,
        pltpu.SemaphoreType.DMA,
        pltpu.SemaphoreType.DMA,
    ],
    compiler_params=pltpu.CompilerParams(use_tc_tiling_on_sc=False),
)
def _spmm_sc(xt_hbm, idx_hbm, val_hbm, out_hbm, idx_v, val_v, rows_v, out_v,
             sem_g0, sem_g1, sem_o0, sem_o1):
    wid = lax.axis_index("s") * NC + lax.axis_index("c")
    row_base = wid * ROWS_PER_W
    grp_base = wid * GROWS_PER_W
    sems_g = (sem_g0, sem_g1)
    sems_o = (sem_o0, sem_o1)

    pltpu.sync_copy(idx_hbm.at[pl.ds(grp_base, GROWS_PER_W)], idx_v)
    pltpu.sync_copy(val_hbm.at[pl.ds(row_base, ROWS_PER_W)], val_v)

    def fire(u, buf):
        for h in range(GPER):
            pltpu.async_copy(xt_hbm.at[idx_v.at[u * GPER + h]],
                             rows_v.at[buf, pl.ds(h * 128, 128)],
                             sems_g[buf])

    def drain_g(buf):
        for h in range(GPER):
            pltpu.make_async_copy(xt_hbm.at[idx_v.at[0]],
                                  rows_v.at[buf, pl.ds(h * 128, 128)],
                                  sems_g[buf]).wait()

    def compute(u, buf):
        def col(cc, carry):
            vv = val_v[u * UNIT + cc, :]
            accs = [jnp.zeros((16,), jnp.float32) for _ in range(BATCH // 16)]
            for k in range(CONN):
                s = vv[k]
                for j in range(BATCH // 16):
                    accs[j] = accs[j] + s * rows_v[buf, cc * CONN + k,
                                                   pl.ds(j * 16, 16)]
            for j in range(BATCH // 16):
                out_v[buf, cc, pl.ds(j * 16, 16)] = accs[j]
            return carry

        lax.fori_loop(0, UNIT, col, 0)

    def write_out(u, buf):
        pltpu.async_copy(out_v.at[buf],
                         out_hbm.at[pl.ds(row_base + u * UNIT, UNIT)],
                         sems_o[buf])

    def wait_out(buf):
        pltpu.make_async_copy(out_v.at[buf],
                              out_hbm.at[pl.ds(row_base, UNIT)],
                              sems_o[buf]).wait()

    fire(0, 0)
    fire(1, 1)
    drain_g(0)
    compute(0, 0)
    fire(2, 0)
    write_out(0, 0)
    drain_g(1)
    compute(1, 1)
    fire(3, 1)
    write_out(1, 1)

    def pair(t, carry):
        u0 = t * 2
        drain_g(0)
        wait_out(0)
        compute(u0, 0)
        fire(u0 + 2, 0)
        write_out(u0, 0)
        drain_g(1)
        wait_out(1)
        compute(u0 + 1, 1)
        fire(u0 + 3, 1)
        write_out(u0 + 1, 1)
        return carry

    lax.fori_loop(1, NUNIT // 2 - 1, pair, 0)

    drain_g(0)
    wait_out(0)
    compute(NUNIT - 2, 0)
    write_out(NUNIT - 2, 0)
    drain_g(1)
    wait_out(1)
    compute(NUNIT - 1, 1)
    write_out(NUNIT - 1, 1)
    wait_out(0)
    wait_out(1)


def kernel(x, indices, values):
    xt = x.T
    idx2d = indices[1].reshape(CONN, OUT_SIZE).T.reshape(-1, 128)
    valt = values.reshape(CONN, OUT_SIZE).T
    outt = _spmm_sc(xt, idx2d, valt)
    return outt.T


# speedup vs baseline: 29.7233x; 1.4195x over previous
import functools

import jax
import jax.numpy as jnp
from jax import lax
from jax.experimental import pallas as pl
from jax.experimental.pallas import tpu as pltpu
from jax.experimental.pallas import tpu_sc as plsc

IN_SIZE = 65536
OUT_SIZE = 65536
CONN = 16
BATCH = 64

NC = 2
NS = 16
NW = NC * NS
ROWS_PER_W = OUT_SIZE // NW
UNIT = 16
NUNIT = ROWS_PER_W // UNIT

_mesh = plsc.VectorSubcoreMesh(core_axis_name="c", subcore_axis_name="s")


@functools.partial(
    pl.kernel,
    mesh=_mesh,
    out_type=jax.ShapeDtypeStruct((OUT_SIZE, BATCH), jnp.float32),
    scratch_types=[
        pltpu.VMEM((CONN, ROWS_PER_W), jnp.int32),
        pltpu.VMEM((CONN, ROWS_PER_W), jnp.float32),
        pltpu.VMEM((2, CONN * UNIT, BATCH), jnp.float32),
        pltpu.VMEM((2, UNIT, BATCH), jnp.float32),
        pltpu.SemaphoreType.DMA,
        pltpu.SemaphoreType.DMA,
        pltpu.SemaphoreType.DMA,
        pltpu.SemaphoreType.DMA,
    ],
    compiler_params=pltpu.CompilerParams(use_tc_tiling_on_sc=False,
                                         needs_layout_passes=False),
)
def _spmm_sc(xt_hbm, idx_hbm, val_hbm, out_hbm, idx_v, val_v, rows_v, out_v,
             sem_g0, sem_g1, sem_o0, sem_o1):
    wid = lax.axis_index("s") * NC + lax.axis_index("c")
    row_base = wid * ROWS_PER_W
    sems_g = (sem_g0, sem_g1)
    sems_o = (sem_o0, sem_o1)

    pltpu.sync_copy(idx_hbm.at[:, pl.ds(row_base, ROWS_PER_W)], idx_v)
    pltpu.sync_copy(val_hbm.at[:, pl.ds(row_base, ROWS_PER_W)], val_v)

    kiota = lax.iota(jnp.int32, 16)

    def fire(u, buf):
        for h in range(CONN):
            pltpu.async_copy(xt_hbm.at[idx_v.at[h, pl.ds(u * UNIT, UNIT)]],
                             rows_v.at[buf, pl.ds(h * UNIT, UNIT)],
                             sems_g[buf])

    def drain_g(buf):
        for h in range(CONN):
            pltpu.make_async_copy(xt_hbm.at[idx_v.at[0, pl.ds(0, UNIT)]],
                                  rows_v.at[buf, pl.ds(h * UNIT, UNIT)],
                                  sems_g[buf]).wait()

    def compute(u, buf):
        def col(cc, carry):
            vv = plsc.load_gather(val_v, [kiota,
                                          jnp.full((16,), u * UNIT + cc,
                                                   jnp.int32)])
            accs = [jnp.zeros((16,), jnp.float32) for _ in range(BATCH // 16)]
            for k in range(CONN):
                s = vv[k]
                for j in range(BATCH // 16):
                    accs[j] = accs[j] + s * rows_v[buf, k * UNIT + cc,
                                                   pl.ds(j * 16, 16)]
            for j in range(BATCH // 16):
                out_v[buf, cc, pl.ds(j * 16, 16)] = accs[j]
            return carry

        lax.fori_loop(0, UNIT, col, 0)

    def write_out(u, buf):
        pltpu.async_copy(out_v.at[buf],
                         out_hbm.at[pl.ds(row_base + u * UNIT, UNIT)],
                         sems_o[buf])

    def wait_out(buf):
        pltpu.make_async_copy(out_v.at[buf],
                              out_hbm.at[pl.ds(row_base, UNIT)],
                              sems_o[buf]).wait()

    fire(0, 0)
    fire(1, 1)
    drain_g(0)
    compute(0, 0)
    fire(2, 0)
    write_out(0, 0)
    drain_g(1)
    compute(1, 1)
    fire(3, 1)
    write_out(1, 1)

    def pair(t, carry):
        u0 = t * 2
        drain_g(0)
        wait_out(0)
        compute(u0, 0)
        fire(u0 + 2, 0)
        write_out(u0, 0)
        drain_g(1)
        wait_out(1)
        compute(u0 + 1, 1)
        fire(u0 + 3, 1)
        write_out(u0 + 1, 1)
        return carry

    lax.fori_loop(1, NUNIT // 2 - 1, pair, 0)

    drain_g(0)
    wait_out(0)
    compute(NUNIT - 2, 0)
    write_out(NUNIT - 2, 0)
    drain_g(1)
    wait_out(1)
    compute(NUNIT - 1, 1)
    write_out(NUNIT - 1, 1)
    wait_out(0)
    wait_out(1)


def kernel(x, indices, values):
    xt = x.T
    idx2d = indices[1].reshape(CONN, OUT_SIZE)
    val2d = values.reshape(CONN, OUT_SIZE)
    outt = _spmm_sc(xt, idx2d, val2d)
    return outt.T
